# Initial kernel scaffold; baseline (speedup 1.0000x reference)
import jax
import jax.numpy as jnp
from jax.experimental import pallas as pl


def kernel(net, edge_index, W_mlp2, W_logr, W_s, W1_logr, b1_logr, W1_s, b1_s, W4_s, b4_s):
    raise NotImplementedError("write your pallas kernel here")


# speedup vs baseline: 29.6257x; 29.6257x over previous
import functools

import jax
import jax.numpy as jnp
from jax import lax
from jax.experimental import pallas as pl
from jax.experimental.pallas import tpu as pltpu
from jax.experimental.pallas import tpu_sc as plsc

_F = 128
_N = 10000
_NP = 10240
_B = 64
_NBLK = 512
_E = 160000
_NSUB = 16
_K = 128
_EPAD = _NSUB * 80 * _K
_NCH = _EPAD // (_NSUB * _K)
_DEGCH = _EPAD // (2 * _NSUB * _K)
_ROWS = _NP // _NSUB

@functools.cache
def _mesh():
    return plsc.VectorSubcoreMesh(core_axis_name="c", subcore_axis_name="s",
                                  num_cores=2, num_subcores=_NSUB)


_DC = 96


def _spmm_body(ngroups, x_hbm, srcg_hbm, dstr_hbm, out_hbm,
               sidx, didx, buf, sem, acc):
    cid = lax.axis_index("c")
    sid = lax.axis_index("s")
    pltpu.sync_copy(dstr_hbm.at[sid], didx)
    for p in range(ngroups // 2):
        g = 2 * p + cid
        pltpu.sync_copy(x_hbm.at[pl.ds(g * _NP + sid * _ROWS, _ROWS)],
                        acc.at[pl.ds(sid * _ROWS, _ROWS)])
        pltpu.sync_copy(srcg_hbm.at[g, sid], sidx)
        plsc.subcore_barrier()

        def chunk(j, carry):
            pltpu.async_copy(x_hbm.at[sidx.at[j]], buf, sem).wait()
            pltpu.sync_copy(buf, acc.at[didx.at[j]], add=True)
            return carry

        lax.fori_loop(0, _NCH, chunk, 0)
        plsc.subcore_barrier()
        pltpu.sync_copy(acc.at[pl.ds(sid * _ROWS, _ROWS)],
                        out_hbm.at[pl.ds(g * _NP + sid * _ROWS, _ROWS)])


@functools.cache
def _make_spmm(ngroups):
    return pl.kernel(
        functools.partial(_spmm_body, ngroups),
        out_type=jax.ShapeDtypeStruct((ngroups * _NP, _DC), jnp.float32),
        mesh=_mesh(),
        scratch_types=[
            pltpu.VMEM((_NCH, _K), jnp.int32),
            pltpu.VMEM((_NCH, _K), jnp.int32),
            pltpu.VMEM((_K, _DC), jnp.float32),
            pltpu.SemaphoreType.DMA,
            pltpu.VMEM_SHARED((_NP, _DC), jnp.float32),
        ],
        compiler_params=pltpu.CompilerParams(use_tc_tiling_on_sc=False),
    )


def _deg_body(zt_hbm, ones_hbm, dstr_hbm, out_hbm, ones_v, didx, acc):
    cid = lax.axis_index("c")
    sid = lax.axis_index("s")
    pltpu.sync_copy(zt_hbm, acc.at[pl.ds(sid * _ROWS, _ROWS)])
    pltpu.sync_copy(ones_hbm, ones_v)
    pltpu.sync_copy(dstr_hbm.at[cid, sid], didx)
    plsc.subcore_barrier()

    def chunk(j, carry):
        pltpu.sync_copy(ones_v, acc.at[didx.at[j]], add=True)
        return carry

    lax.fori_loop(0, _DEGCH, chunk, 0)
    plsc.subcore_barrier()
    pltpu.sync_copy(acc.at[pl.ds(sid * _ROWS, _ROWS)],
                    out_hbm.at[pl.ds(cid * _NP + sid * _ROWS, _ROWS)])


@functools.cache
def _deg_kernel_fn():
  return pl.kernel(
    _deg_body,
    out_type=jax.ShapeDtypeStruct((2 * _NP, 16), jnp.float32),
    mesh=_mesh(),
    scratch_types=[
        pltpu.VMEM((_K, 16), jnp.float32),
        pltpu.VMEM((_DEGCH, _K), jnp.int32),
        pltpu.VMEM_SHARED((_NP, 16), jnp.float32),
    ],
    compiler_params=pltpu.CompilerParams(use_tc_tiling_on_sc=False),
)


def _h_body(wm2_ref, nett_ref, ht_ref):
    dims = (((0,), (0,)), ((), ()))
    ht_ref[...] = jnp.tanh(
        lax.dot_general(wm2_ref[...], nett_ref[...], dims,
                        preferred_element_type=jnp.float32))


def _dinv_body(dd_ref, di_ref):
    deg = 1.0 + dd_ref[0:_NP, 0:1] + dd_ref[_NP:2 * _NP, 0:1]
    di_ref[...] = lax.rsqrt(deg)


def _dense1_body(wl_ref, ws_ref, h1t_ref, h2t_ref, d3_ref, d6_ref,
                 xl_ref, xs_ref):
    dims = (((0,), (0,)), ((), ()))
    xl = lax.dot_general(wl_ref[...], h1t_ref[...], dims,
                         preferred_element_type=jnp.float32)
    xs = lax.dot_general(ws_ref[...], h2t_ref[...], dims,
                         preferred_element_type=jnp.float32)
    xl_ref[...] = d3_ref[...] * jnp.tanh(xl)
    xs_ref[...] = d6_ref[...] * jnp.tanh(xs)


def _mix_body(do_tanh, dinv_out, scale, y_ref, d_ref, m_ref, b_ref, o_ref):
    d = d_ref[...]
    z = lax.dot_general(d * y_ref[...], m_ref[...], (((1,), (0,)), ((), ())),
                        preferred_element_type=jnp.float32) + b_ref[...]
    if do_tanh:
        z = jnp.tanh(z)
    if dinv_out:
        z = d * z
    if scale != 1.0:
        z = z * scale
    o_ref[...] = z


def _mix(y, dinv, M, brow, do_tanh=False, dinv_out=False, scale=1.0):
    cb = y.shape[1]
    grid = _NP // _NBLK
    return pl.pallas_call(
        functools.partial(_mix_body, do_tanh, dinv_out, scale),
        grid=(grid,),
        in_specs=[
            pl.BlockSpec((_NBLK, cb), lambda i: (i, 0)),
            pl.BlockSpec((_NBLK, 1), lambda i: (i, 0)),
            pl.BlockSpec((cb, cb), lambda i: (0, 0)),
            pl.BlockSpec((1, cb), lambda i: (0, 0)),
        ],
        out_specs=pl.BlockSpec((_NBLK, cb), lambda i: (i, 0)),
        out_shape=jax.ShapeDtypeStruct((_NP, cb), jnp.float32),
    )(y, dinv, M, brow)


def _dense1(W_logr, W_s, h1t, h2t, dinv3, dinv6):
    grid = _NP // _NBLK
    return pl.pallas_call(
        _dense1_body,
        grid=(grid,),
        in_specs=[
            pl.BlockSpec((_F, _NBLK * 3), lambda i: (0, i)),
            pl.BlockSpec((_F, _NBLK * 6), lambda i: (0, i)),
            pl.BlockSpec((_F, _B), lambda i: (0, 0)),
            pl.BlockSpec((_F, _B), lambda i: (0, 0)),
            pl.BlockSpec((_NBLK * 3, 1), lambda i: (i, 0)),
            pl.BlockSpec((_NBLK * 6, 1), lambda i: (i, 0)),
        ],
        out_specs=[
            pl.BlockSpec((_NBLK * 3, _B), lambda i: (i, 0)),
            pl.BlockSpec((_NBLK * 6, _B), lambda i: (i, 0)),
        ],
        out_shape=[
            jax.ShapeDtypeStruct((_NP * 3, _B), jnp.float32),
            jax.ShapeDtypeStruct((_NP * 6, _B), jnp.float32),
        ],
    )(W_logr, W_s, h1t, h2t, dinv3, dinv6)


def _spmm_apply(x2d, srcg, dstr):
    ng = x2d.shape[1] // _DC
    xcat = jnp.concatenate([x2d[:, i * _DC:(i + 1) * _DC] for i in range(ng)],
                           axis=0)
    ycat = _make_spmm(ng)(xcat, srcg, dstr)
    return jnp.concatenate([ycat[i * _NP:(i + 1) * _NP] for i in range(ng)],
                           axis=1)


def kernel(net, edge_index, W_mlp2, W_logr, W_s, W1_logr, b1_logr, W1_s, b1_s,
           W4_s, b4_s):
    src = edge_index[0].astype(jnp.int32)
    dst = edge_index[1].astype(jnp.int32)

    padlen = _EPAD - _E
    srcp = jnp.concatenate([src, jnp.full((padlen,), _N, jnp.int32)])
    dstp = jnp.concatenate([dst, jnp.full((padlen,), _N, jnp.int32)])
    srcr = srcp.reshape(_NSUB, _NCH, _K)
    srcg = jnp.stack([srcr + g * _NP for g in range(4)])
    dstr = dstp.reshape(_NSUB, _NCH, _K)
    dstr_deg = dstp.reshape(2, _NSUB, _DEGCH, _K)
    zt = jnp.zeros((_ROWS, 16), jnp.float32)
    ones_t = jnp.zeros((_K, 16), jnp.float32).at[:, 0].set(1.0)

    dd = _deg_kernel_fn()(zt, ones_t, dstr_deg)
    dinv = pl.pallas_call(
        _dinv_body,
        in_specs=[pl.BlockSpec((2 * _NP, 16), lambda: (0, 0))],
        out_specs=pl.BlockSpec((_NP, 1), lambda: (0, 0)),
        out_shape=jax.ShapeDtypeStruct((_NP, 1), jnp.float32),
    )(dd)
    dinv3 = jnp.repeat(dinv[:, 0], 3)[:, None]
    dinv6 = jnp.repeat(dinv[:, 0], 6)[:, None]

    W_logr = jnp.pad(W_logr.reshape(_F, _N, 3),
                     ((0, 0), (0, _NP - _N), (0, 0))).reshape(_F, _NP * 3)
    W_s = jnp.pad(W_s.reshape(_F, _N, 6),
                  ((0, 0), (0, _NP - _N), (0, 0))).reshape(_F, _NP * 6)
    ht = pl.pallas_call(
        _h_body,
        in_specs=[
            pl.BlockSpec((2 * _F, 2 * _F), lambda: (0, 0)),
            pl.BlockSpec((2 * _F, _B), lambda: (0, 0)),
        ],
        out_specs=pl.BlockSpec((2 * _F, _B), lambda: (0, 0)),
        out_shape=jax.ShapeDtypeStruct((2 * _F, _B), jnp.float32),
    )(W_mlp2, net.T)
    xl, xs = _dense1(W_logr, W_s, ht[:_F], ht[_F:], dinv3, dinv6)

    eyeB = jnp.eye(_B, dtype=jnp.float32)
    Ml = jnp.kron(W1_logr, eyeB)
    Ms1 = jnp.kron(W1_s, eyeB)
    Ms4 = jnp.kron(W4_s, eyeB)
    bl = jnp.repeat(b1_logr, _B)[None, :]
    bs1 = jnp.repeat(b1_s, _B)[None, :]
    bs4 = jnp.repeat(b4_s, _B)[None, :]

    yl = _spmm_apply(xl.reshape(_NP, 192), srcg[:2], dstr)
    logr_f = _mix(yl, dinv, Ml, bl, scale=4.0)

    ys1 = _spmm_apply(xs.reshape(_NP, 384), srcg, dstr)
    x2 = _mix(ys1, dinv, Ms1, bs1, do_tanh=True, dinv_out=True)
    ys2 = _spmm_apply(x2, srcg, dstr)
    s_f = _mix(ys2, dinv, Ms4, bs4, scale=50.0)

    cat = jnp.concatenate([logr_f.reshape(_NP, 3, _B)[:_N],
                           s_f.reshape(_NP, 6, _B)[:_N]], axis=1)
    return jnp.transpose(cat, (2, 0, 1))


# speedup vs baseline: 31.6688x; 1.0690x over previous
import functools

import jax
import jax.numpy as jnp
from jax import lax
from jax.experimental import pallas as pl
from jax.experimental.pallas import tpu as pltpu
from jax.experimental.pallas import tpu_sc as plsc

_F = 128
_N = 10000
_NP = 10240
_B = 64
_NBLK = 512
_E = 160000
_NSUB = 16
_K = 128
_EPAD = _NSUB * 80 * _K
_NCH = _EPAD // (_NSUB * _K)
_DEGCH = _EPAD // (2 * _NSUB * _K)
_ROWS = _NP // _NSUB

@functools.cache
def _mesh():
    return plsc.VectorSubcoreMesh(core_axis_name="c", subcore_axis_name="s",
                                  num_cores=2, num_subcores=_NSUB)


_DC = 96


def _spmm_body(ngroups, x_hbm, srcg_hbm, dstr_hbm, out_hbm,
               sidx, didx, buf, buf2, sem, sem2, acc):
    cid = lax.axis_index("c")
    sid = lax.axis_index("s")
    pltpu.sync_copy(dstr_hbm.at[sid], didx)
    for p in range(ngroups // 2):
        g = 2 * p + cid
        pltpu.sync_copy(x_hbm.at[pl.ds(g * _NP + sid * _ROWS, _ROWS)],
                        acc.at[pl.ds(sid * _ROWS, _ROWS)])
        pltpu.sync_copy(srcg_hbm.at[g, sid], sidx)
        plsc.subcore_barrier()

        def chunk2(i, carry):
            j = 2 * i
            cp0 = pltpu.async_copy(x_hbm.at[sidx.at[j]], buf, sem)
            cp1 = pltpu.async_copy(x_hbm.at[sidx.at[j + 1]], buf2, sem2)
            cp0.wait()
            pltpu.sync_copy(buf, acc.at[didx.at[j]], add=True)
            cp1.wait()
            pltpu.sync_copy(buf2, acc.at[didx.at[j + 1]], add=True)
            return carry

        lax.fori_loop(0, _NCH // 2, chunk2, 0)
        plsc.subcore_barrier()
        pltpu.sync_copy(acc.at[pl.ds(sid * _ROWS, _ROWS)],
                        out_hbm.at[pl.ds(g * _NP + sid * _ROWS, _ROWS)])


@functools.cache
def _make_spmm(ngroups):
    return pl.kernel(
        functools.partial(_spmm_body, ngroups),
        out_type=jax.ShapeDtypeStruct((ngroups * _NP, _DC), jnp.float32),
        mesh=_mesh(),
        scratch_types=[
            pltpu.VMEM((_NCH, _K), jnp.int32),
            pltpu.VMEM((_NCH, _K), jnp.int32),
            pltpu.VMEM((_K, _DC), jnp.float32),
            pltpu.VMEM((_K, _DC), jnp.float32),
            pltpu.SemaphoreType.DMA,
            pltpu.SemaphoreType.DMA,
            pltpu.VMEM_SHARED((_NP, _DC), jnp.float32),
        ],
        compiler_params=pltpu.CompilerParams(use_tc_tiling_on_sc=False),
    )


def _deg_body(zt_hbm, ones_hbm, dstr_hbm, out_hbm, ones_v, didx, acc):
    cid = lax.axis_index("c")
    sid = lax.axis_index("s")
    pltpu.sync_copy(zt_hbm, acc.at[pl.ds(sid * _ROWS, _ROWS)])
    pltpu.sync_copy(ones_hbm, ones_v)
    pltpu.sync_copy(dstr_hbm.at[cid, sid], didx)
    plsc.subcore_barrier()

    def chunk(j, carry):
        pltpu.sync_copy(ones_v, acc.at[didx.at[j]], add=True)
        return carry

    lax.fori_loop(0, _DEGCH, chunk, 0)
    plsc.subcore_barrier()
    pltpu.sync_copy(acc.at[pl.ds(sid * _ROWS, _ROWS)],
                    out_hbm.at[pl.ds(cid * _NP + sid * _ROWS, _ROWS)])


@functools.cache
def _deg_kernel_fn():
  return pl.kernel(
    _deg_body,
    out_type=jax.ShapeDtypeStruct((2 * _NP, 16), jnp.float32),
    mesh=_mesh(),
    scratch_types=[
        pltpu.VMEM((_K, 16), jnp.float32),
        pltpu.VMEM((_DEGCH, _K), jnp.int32),
        pltpu.VMEM_SHARED((_NP, 16), jnp.float32),
    ],
    compiler_params=pltpu.CompilerParams(use_tc_tiling_on_sc=False),
)


def _h_body(wm2_ref, nett_ref, ht_ref):
    dims = (((0,), (0,)), ((), ()))
    ht_ref[...] = jnp.tanh(
        lax.dot_general(wm2_ref[...], nett_ref[...], dims,
                        preferred_element_type=jnp.float32))


def _dinv_body(dd_ref, di_ref):
    deg = 1.0 + dd_ref[0:_NP, 0:1] + dd_ref[_NP:2 * _NP, 0:1]
    di_ref[...] = lax.rsqrt(deg)


def _dense1_body(wl_ref, ws_ref, h1t_ref, h2t_ref, d3_ref, d6_ref,
                 xl_ref, xs_ref):
    dims = (((0,), (0,)), ((), ()))
    xl = lax.dot_general(wl_ref[...], h1t_ref[...], dims,
                         preferred_element_type=jnp.float32)
    xs = lax.dot_general(ws_ref[...], h2t_ref[...], dims,
                         preferred_element_type=jnp.float32)
    xl_ref[...] = d3_ref[...] * jnp.tanh(xl)
    xs_ref[...] = d6_ref[...] * jnp.tanh(xs)


def _mix_body(do_tanh, dinv_out, scale, y_ref, d_ref, m_ref, b_ref, o_ref):
    d = d_ref[...]
    z = lax.dot_general(d * y_ref[...], m_ref[...], (((1,), (0,)), ((), ())),
                        preferred_element_type=jnp.float32) + b_ref[...]
    if do_tanh:
        z = jnp.tanh(z)
    if dinv_out:
        z = d * z
    if scale != 1.0:
        z = z * scale
    o_ref[...] = z


def _mix(y, dinv, M, brow, do_tanh=False, dinv_out=False, scale=1.0):
    cb = y.shape[1]
    grid = _NP // _NBLK
    return pl.pallas_call(
        functools.partial(_mix_body, do_tanh, dinv_out, scale),
        grid=(grid,),
        in_specs=[
            pl.BlockSpec((_NBLK, cb), lambda i: (i, 0)),
            pl.BlockSpec((_NBLK, 1), lambda i: (i, 0)),
            pl.BlockSpec((cb, cb), lambda i: (0, 0)),
            pl.BlockSpec((1, cb), lambda i: (0, 0)),
        ],
        out_specs=pl.BlockSpec((_NBLK, cb), lambda i: (i, 0)),
        out_shape=jax.ShapeDtypeStruct((_NP, cb), jnp.float32),
    )(y, dinv, M, brow)


def _dense1(W_logr, W_s, h1t, h2t, dinv3, dinv6):
    grid = _NP // _NBLK
    return pl.pallas_call(
        _dense1_body,
        grid=(grid,),
        in_specs=[
            pl.BlockSpec((_F, _NBLK * 3), lambda i: (0, i)),
            pl.BlockSpec((_F, _NBLK * 6), lambda i: (0, i)),
            pl.BlockSpec((_F, _B), lambda i: (0, 0)),
            pl.BlockSpec((_F, _B), lambda i: (0, 0)),
            pl.BlockSpec((_NBLK * 3, 1), lambda i: (i, 0)),
            pl.BlockSpec((_NBLK * 6, 1), lambda i: (i, 0)),
        ],
        out_specs=[
            pl.BlockSpec((_NBLK * 3, _B), lambda i: (i, 0)),
            pl.BlockSpec((_NBLK * 6, _B), lambda i: (i, 0)),
        ],
        out_shape=[
            jax.ShapeDtypeStruct((_NP * 3, _B), jnp.float32),
            jax.ShapeDtypeStruct((_NP * 6, _B), jnp.float32),
        ],
    )(W_logr, W_s, h1t, h2t, dinv3, dinv6)


def _spmm_apply(x2d, srcg, dstr):
    ng = x2d.shape[1] // _DC
    xcat = jnp.concatenate([x2d[:, i * _DC:(i + 1) * _DC] for i in range(ng)],
                           axis=0)
    ycat = _make_spmm(ng)(xcat, srcg, dstr)
    return jnp.concatenate([ycat[i * _NP:(i + 1) * _NP] for i in range(ng)],
                           axis=1)


def kernel(net, edge_index, W_mlp2, W_logr, W_s, W1_logr, b1_logr, W1_s, b1_s,
           W4_s, b4_s):
    src = edge_index[0].astype(jnp.int32)
    dst = edge_index[1].astype(jnp.int32)

    padlen = _EPAD - _E
    srcp = jnp.concatenate([src, jnp.full((padlen,), _N, jnp.int32)])
    dstp = jnp.concatenate([dst, jnp.full((padlen,), _N, jnp.int32)])
    srcr = srcp.reshape(_NSUB, _NCH, _K)
    srcg = jnp.stack([srcr + g * _NP for g in range(4)])
    dstr = dstp.reshape(_NSUB, _NCH, _K)
    dstr_deg = dstp.reshape(2, _NSUB, _DEGCH, _K)
    zt = jnp.zeros((_ROWS, 16), jnp.float32)
    ones_t = jnp.zeros((_K, 16), jnp.float32).at[:, 0].set(1.0)

    dd = _deg_kernel_fn()(zt, ones_t, dstr_deg)
    dinv = pl.pallas_call(
        _dinv_body,
        in_specs=[pl.BlockSpec((2 * _NP, 16), lambda: (0, 0))],
        out_specs=pl.BlockSpec((_NP, 1), lambda: (0, 0)),
        out_shape=jax.ShapeDtypeStruct((_NP, 1), jnp.float32),
    )(dd)
    dinv3 = jnp.repeat(dinv[:, 0], 3)[:, None]
    dinv6 = jnp.repeat(dinv[:, 0], 6)[:, None]

    W_logr = jnp.pad(W_logr.reshape(_F, _N, 3),
                     ((0, 0), (0, _NP - _N), (0, 0))).reshape(_F, _NP * 3)
    W_s = jnp.pad(W_s.reshape(_F, _N, 6),
                  ((0, 0), (0, _NP - _N), (0, 0))).reshape(_F, _NP * 6)
    ht = pl.pallas_call(
        _h_body,
        in_specs=[
            pl.BlockSpec((2 * _F, 2 * _F), lambda: (0, 0)),
            pl.BlockSpec((2 * _F, _B), lambda: (0, 0)),
        ],
        out_specs=pl.BlockSpec((2 * _F, _B), lambda: (0, 0)),
        out_shape=jax.ShapeDtypeStruct((2 * _F, _B), jnp.float32),
    )(W_mlp2, net.T)
    xl, xs = _dense1(W_logr, W_s, ht[:_F], ht[_F:], dinv3, dinv6)

    eyeB = jnp.eye(_B, dtype=jnp.float32)
    Ml = jnp.kron(W1_logr, eyeB)
    Ms1 = jnp.kron(W1_s, eyeB)
    Ms4 = jnp.kron(W4_s, eyeB)
    bl = jnp.repeat(b1_logr, _B)[None, :]
    bs1 = jnp.repeat(b1_s, _B)[None, :]
    bs4 = jnp.repeat(b4_s, _B)[None, :]

    yl = _spmm_apply(xl.reshape(_NP, 192), srcg[:2], dstr)
    logr_f = _mix(yl, dinv, Ml, bl, scale=4.0)

    ys1 = _spmm_apply(xs.reshape(_NP, 384), srcg, dstr)
    x2 = _mix(ys1, dinv, Ms1, bs1, do_tanh=True, dinv_out=True)
    ys2 = _spmm_apply(x2, srcg, dstr)
    s_f = _mix(ys2, dinv, Ms4, bs4, scale=50.0)

    cat = jnp.concatenate([logr_f.reshape(_NP, 3, _B)[:_N],
                           s_f.reshape(_NP, 6, _B)[:_N]], axis=1)
    return jnp.transpose(cat, (2, 0, 1))


# speedup vs baseline: 32.5982x; 1.0293x over previous
import functools

import jax
import jax.numpy as jnp
from jax import lax
from jax.experimental import pallas as pl
from jax.experimental.pallas import tpu as pltpu
from jax.experimental.pallas import tpu_sc as plsc

_F = 128
_N = 10000
_NP = 10240
_B = 64
_NBLK = 512
_E = 160000
_NSUB = 16
_K = 128
_EPAD = _NSUB * 80 * _K
_NCH = _EPAD // (_NSUB * _K)
_DEGCH = _EPAD // (2 * _NSUB * _K)
_ROWS = _NP // _NSUB

@functools.cache
def _mesh():
    return plsc.VectorSubcoreMesh(core_axis_name="c", subcore_axis_name="s",
                                  num_cores=2, num_subcores=_NSUB)


_DC = 96


def _spmm_body(ngroups, x_hbm, srcg_hbm, dstr_hbm, out_hbm,
               sidx, didx, b0, b1, b2, b3, s0, s1, s2, s3, acc):
    bufs = (b0, b1, b2, b3)
    sems = (s0, s1, s2, s3)
    cid = lax.axis_index("c")
    sid = lax.axis_index("s")
    pltpu.sync_copy(dstr_hbm.at[sid], didx)
    for p in range(ngroups // 2):
        g = 2 * p + cid
        pltpu.sync_copy(x_hbm.at[pl.ds(g * _NP + sid * _ROWS, _ROWS)],
                        acc.at[pl.ds(sid * _ROWS, _ROWS)])
        pltpu.sync_copy(srcg_hbm.at[g, sid], sidx)
        plsc.subcore_barrier()

        def chunk4(i, carry):
            j = 4 * i
            cps = [pltpu.async_copy(x_hbm.at[sidx.at[j + t]], bufs[t], sems[t])
                   for t in range(4)]
            for t in range(4):
                cps[t].wait()
                pltpu.sync_copy(bufs[t], acc.at[didx.at[j + t]], add=True)
            return carry

        lax.fori_loop(0, _NCH // 4, chunk4, 0)
        plsc.subcore_barrier()
        pltpu.sync_copy(acc.at[pl.ds(sid * _ROWS, _ROWS)],
                        out_hbm.at[pl.ds(g * _NP + sid * _ROWS, _ROWS)])


@functools.cache
def _make_spmm(ngroups):
    return pl.kernel(
        functools.partial(_spmm_body, ngroups),
        out_type=jax.ShapeDtypeStruct((ngroups * _NP, _DC), jnp.float32),
        mesh=_mesh(),
        scratch_types=[
            pltpu.VMEM((_NCH, _K), jnp.int32),
            pltpu.VMEM((_NCH, _K), jnp.int32),
            pltpu.VMEM((_K, _DC), jnp.float32),
            pltpu.VMEM((_K, _DC), jnp.float32),
            pltpu.VMEM((_K, _DC), jnp.float32),
            pltpu.VMEM((_K, _DC), jnp.float32),
            pltpu.SemaphoreType.DMA,
            pltpu.SemaphoreType.DMA,
            pltpu.SemaphoreType.DMA,
            pltpu.SemaphoreType.DMA,
            pltpu.VMEM_SHARED((_NP, _DC), jnp.float32),
        ],
        compiler_params=pltpu.CompilerParams(use_tc_tiling_on_sc=False),
    )


def _deg_body(zt_hbm, ones_hbm, dstr_hbm, out_hbm, ones_v, didx, acc):
    cid = lax.axis_index("c")
    sid = lax.axis_index("s")
    pltpu.sync_copy(zt_hbm, acc.at[pl.ds(sid * _ROWS, _ROWS)])
    pltpu.sync_copy(ones_hbm, ones_v)
    pltpu.sync_copy(dstr_hbm.at[cid, sid], didx)
    plsc.subcore_barrier()

    def chunk(j, carry):
        pltpu.sync_copy(ones_v, acc.at[didx.at[j]], add=True)
        return carry

    lax.fori_loop(0, _DEGCH, chunk, 0)
    plsc.subcore_barrier()
    pltpu.sync_copy(acc.at[pl.ds(sid * _ROWS, _ROWS)],
                    out_hbm.at[pl.ds(cid * _NP + sid * _ROWS, _ROWS)])


@functools.cache
def _deg_kernel_fn():
  return pl.kernel(
    _deg_body,
    out_type=jax.ShapeDtypeStruct((2 * _NP, 16), jnp.float32),
    mesh=_mesh(),
    scratch_types=[
        pltpu.VMEM((_K, 16), jnp.float32),
        pltpu.VMEM((_DEGCH, _K), jnp.int32),
        pltpu.VMEM_SHARED((_NP, 16), jnp.float32),
    ],
    compiler_params=pltpu.CompilerParams(use_tc_tiling_on_sc=False),
)


def _h_body(wm2_ref, nett_ref, ht_ref):
    dims = (((0,), (0,)), ((), ()))
    ht_ref[...] = jnp.tanh(
        lax.dot_general(wm2_ref[...], nett_ref[...], dims,
                        preferred_element_type=jnp.float32))


def _dinv_body(dd_ref, di_ref):
    deg = 1.0 + dd_ref[0:_NP, 0:1] + dd_ref[_NP:2 * _NP, 0:1]
    di_ref[...] = lax.rsqrt(deg)


def _dense1_body(wl_ref, ws_ref, h1t_ref, h2t_ref, d3_ref, d6_ref,
                 xl_ref, xs_ref):
    dims = (((0,), (0,)), ((), ()))
    xl = lax.dot_general(wl_ref[...], h1t_ref[...], dims,
                         preferred_element_type=jnp.float32)
    xs = lax.dot_general(ws_ref[...], h2t_ref[...], dims,
                         preferred_element_type=jnp.float32)
    xl_ref[...] = d3_ref[...] * jnp.tanh(xl)
    xs_ref[...] = d6_ref[...] * jnp.tanh(xs)


def _mix_body(do_tanh, dinv_out, scale, y_ref, d_ref, m_ref, b_ref, o_ref):
    d = d_ref[...]
    z = lax.dot_general(d * y_ref[...], m_ref[...], (((1,), (0,)), ((), ())),
                        preferred_element_type=jnp.float32) + b_ref[...]
    if do_tanh:
        z = jnp.tanh(z)
    if dinv_out:
        z = d * z
    if scale != 1.0:
        z = z * scale
    o_ref[...] = z


def _mix(y, dinv, M, brow, do_tanh=False, dinv_out=False, scale=1.0):
    cb = y.shape[1]
    grid = _NP // _NBLK
    return pl.pallas_call(
        functools.partial(_mix_body, do_tanh, dinv_out, scale),
        grid=(grid,),
        in_specs=[
            pl.BlockSpec((_NBLK, cb), lambda i: (i, 0)),
            pl.BlockSpec((_NBLK, 1), lambda i: (i, 0)),
            pl.BlockSpec((cb, cb), lambda i: (0, 0)),
            pl.BlockSpec((1, cb), lambda i: (0, 0)),
        ],
        out_specs=pl.BlockSpec((_NBLK, cb), lambda i: (i, 0)),
        out_shape=jax.ShapeDtypeStruct((_NP, cb), jnp.float32),
    )(y, dinv, M, brow)


def _dense1(W_logr, W_s, h1t, h2t, dinv3, dinv6):
    grid = _NP // _NBLK
    return pl.pallas_call(
        _dense1_body,
        grid=(grid,),
        in_specs=[
            pl.BlockSpec((_F, _NBLK * 3), lambda i: (0, i)),
            pl.BlockSpec((_F, _NBLK * 6), lambda i: (0, i)),
            pl.BlockSpec((_F, _B), lambda i: (0, 0)),
            pl.BlockSpec((_F, _B), lambda i: (0, 0)),
            pl.BlockSpec((_NBLK * 3, 1), lambda i: (i, 0)),
            pl.BlockSpec((_NBLK * 6, 1), lambda i: (i, 0)),
        ],
        out_specs=[
            pl.BlockSpec((_NBLK * 3, _B), lambda i: (i, 0)),
            pl.BlockSpec((_NBLK * 6, _B), lambda i: (i, 0)),
        ],
        out_shape=[
            jax.ShapeDtypeStruct((_NP * 3, _B), jnp.float32),
            jax.ShapeDtypeStruct((_NP * 6, _B), jnp.float32),
        ],
    )(W_logr, W_s, h1t, h2t, dinv3, dinv6)


def _spmm_apply(x2d, srcg, dstr):
    ng = x2d.shape[1] // _DC
    xcat = jnp.concatenate([x2d[:, i * _DC:(i + 1) * _DC] for i in range(ng)],
                           axis=0)
    ycat = _make_spmm(ng)(xcat, srcg, dstr)
    return jnp.concatenate([ycat[i * _NP:(i + 1) * _NP] for i in range(ng)],
                           axis=1)


def kernel(net, edge_index, W_mlp2, W_logr, W_s, W1_logr, b1_logr, W1_s, b1_s,
           W4_s, b4_s):
    src = edge_index[0].astype(jnp.int32)
    dst = edge_index[1].astype(jnp.int32)

    padlen = _EPAD - _E
    srcp = jnp.concatenate([src, jnp.full((padlen,), _N, jnp.int32)])
    dstp = jnp.concatenate([dst, jnp.full((padlen,), _N, jnp.int32)])
    srcr = srcp.reshape(_NSUB, _NCH, _K)
    srcg = jnp.stack([srcr + g * _NP for g in range(4)])
    dstr = dstp.reshape(_NSUB, _NCH, _K)
    dstr_deg = dstp.reshape(2, _NSUB, _DEGCH, _K)
    zt = jnp.zeros((_ROWS, 16), jnp.float32)
    ones_t = jnp.zeros((_K, 16), jnp.float32).at[:, 0].set(1.0)

    dd = _deg_kernel_fn()(zt, ones_t, dstr_deg)
    dinv = pl.pallas_call(
        _dinv_body,
        in_specs=[pl.BlockSpec((2 * _NP, 16), lambda: (0, 0))],
        out_specs=pl.BlockSpec((_NP, 1), lambda: (0, 0)),
        out_shape=jax.ShapeDtypeStruct((_NP, 1), jnp.float32),
    )(dd)
    dinv3 = jnp.repeat(dinv[:, 0], 3)[:, None]
    dinv6 = jnp.repeat(dinv[:, 0], 6)[:, None]

    W_logr = jnp.pad(W_logr.reshape(_F, _N, 3),
                     ((0, 0), (0, _NP - _N), (0, 0))).reshape(_F, _NP * 3)
    W_s = jnp.pad(W_s.reshape(_F, _N, 6),
                  ((0, 0), (0, _NP - _N), (0, 0))).reshape(_F, _NP * 6)
    ht = pl.pallas_call(
        _h_body,
        in_specs=[
            pl.BlockSpec((2 * _F, 2 * _F), lambda: (0, 0)),
            pl.BlockSpec((2 * _F, _B), lambda: (0, 0)),
        ],
        out_specs=pl.BlockSpec((2 * _F, _B), lambda: (0, 0)),
        out_shape=jax.ShapeDtypeStruct((2 * _F, _B), jnp.float32),
    )(W_mlp2, net.T)
    xl, xs = _dense1(W_logr, W_s, ht[:_F], ht[_F:], dinv3, dinv6)

    eyeB = jnp.eye(_B, dtype=jnp.float32)
    Ml = jnp.kron(W1_logr, eyeB)
    Ms1 = jnp.kron(W1_s, eyeB)
    Ms4 = jnp.kron(W4_s, eyeB)
    bl = jnp.repeat(b1_logr, _B)[None, :]
    bs1 = jnp.repeat(b1_s, _B)[None, :]
    bs4 = jnp.repeat(b4_s, _B)[None, :]

    yl = _spmm_apply(xl.reshape(_NP, 192), srcg[:2], dstr)
    logr_f = _mix(yl, dinv, Ml, bl, scale=4.0)

    ys1 = _spmm_apply(xs.reshape(_NP, 384), srcg, dstr)
    x2 = _mix(ys1, dinv, Ms1, bs1, do_tanh=True, dinv_out=True)
    ys2 = _spmm_apply(x2, srcg, dstr)
    s_f = _mix(ys2, dinv, Ms4, bs4, scale=50.0)

    cat = jnp.concatenate([logr_f.reshape(_NP, 3, _B)[:_N],
                           s_f.reshape(_NP, 6, _B)[:_N]], axis=1)
    return jnp.transpose(cat, (2, 0, 1))
